# Initial kernel scaffold; baseline (speedup 1.0000x reference)
import jax
import jax.numpy as jnp
from jax.experimental import pallas as pl


def kernel(prediction, target):
    raise NotImplementedError("write your pallas kernel here")


# speedup vs baseline: 1.0157x; 1.0157x over previous
import functools

import jax
import jax.numpy as jnp
from jax import lax
from jax.experimental import pallas as pl
from jax.experimental.pallas import tpu as pltpu
from jax.experimental.pallas import tpu_sc as plsc

B = 128
V = 100000
TOPK_K = 5
GRAN = 128
CHUNK = 8192
NSTEPS = -(-V // CHUNK)

NW_ACTIVE = 16
B_PER_W = B // NW_ACTIVE


def _sc_gather(table, idx):
    mesh = plsc.VectorSubcoreMesh(core_axis_name="c", subcore_axis_name="s")

    @functools.partial(
        pl.kernel,
        mesh=mesh,
        out_type=jax.ShapeDtypeStruct((B, GRAN), jnp.float32),
        scratch_types=[
            pltpu.VMEM((B_PER_W,), jnp.int32),
            pltpu.VMEM((B_PER_W, GRAN), jnp.float32),
            pltpu.SemaphoreType.DMA,
        ],
    )
    def k(table_hbm, idx_hbm, out_hbm, idx_v, rows_v, sem):
        wid = lax.axis_index("s") * 2 + lax.axis_index("c")

        @pl.when(wid < NW_ACTIVE)
        def _():
            base = wid * B_PER_W
            pltpu.sync_copy(idx_hbm.at[pl.ds(base, B_PER_W)], idx_v)
            pltpu.async_copy(table_hbm.at[idx_v], rows_v, sem).wait()
            pltpu.sync_copy(rows_v, out_hbm.at[pl.ds(base, B_PER_W)])

    return k(table, idx)


def _count_body(t_ref, g_ref, c_ref, p_ref, o_ref, v_ref, acc_ref):
    step = pl.program_id(0)

    @pl.when(step == 0)
    def _():
        lane = lax.broadcasted_iota(jnp.int32, (B, GRAN), 1)
        v_ref[...] = jnp.sum(
            jnp.where(lane == c_ref[...], g_ref[...], 0.0),
            axis=1, keepdims=True)
        acc_ref[...] = jnp.zeros_like(acc_ref)

    p = p_ref[...]
    v = v_ref[...]
    col = step * CHUNK + lax.broadcasted_iota(jnp.int32, p.shape, 1)
    better = (p > v) | ((p == v) & (col < t_ref[...]))

    @pl.when(step < NSTEPS - 1)
    def _():
        acc_ref[...] += jnp.sum(better.astype(jnp.float32), axis=1,
                                keepdims=True)

    @pl.when(step == NSTEPS - 1)
    def _():
        m = better & (col < V)
        acc = acc_ref[...] + jnp.sum(m.astype(jnp.float32), axis=1,
                                     keepdims=True)
        hits = jnp.sum((acc < float(TOPK_K)).astype(jnp.float32),
                       axis=0, keepdims=True)
        o_ref[...] = hits * (100.0 / B)


def _count(prediction, t, g, c):
    return pl.pallas_call(
        _count_body,
        grid=(NSTEPS,),
        in_specs=[
            pl.BlockSpec((B, 1), lambda i: (0, 0)),
            pl.BlockSpec((B, GRAN), lambda i: (0, 0)),
            pl.BlockSpec((B, 1), lambda i: (0, 0)),
            pl.BlockSpec((B, CHUNK), lambda i: (0, i)),
        ],
        out_specs=pl.BlockSpec((1, 1), lambda i: (0, 0)),
        out_shape=jax.ShapeDtypeStruct((1, 1), jnp.float32),
        scratch_shapes=[
            pltpu.VMEM((B, 1), jnp.float32),
            pltpu.VMEM((B, 1), jnp.float32),
        ],
    )(t, g, c, prediction)


def kernel(prediction, target):
    target = target.astype(jnp.int32)
    t = target.reshape(B, 1)
    flat = jnp.arange(B, dtype=jnp.int32) * V + target
    rows = (flat // GRAN).astype(jnp.int32)
    lanes = (flat % GRAN).reshape(B, 1)
    table = prediction.reshape(B * V // GRAN, GRAN)
    g = _sc_gather(table, rows)
    res = _count(prediction, t, g, lanes)
    return res[0, 0]


# speedup vs baseline: 1.5629x; 1.5387x over previous
import functools

import jax
import jax.numpy as jnp
from jax import lax
from jax.experimental import pallas as pl
from jax.experimental.pallas import tpu as pltpu
from jax.experimental.pallas import tpu_sc as plsc

B = 128
V = 100000
TOPK_K = 5
SUBL = 8
LANE = 128
CHUNK = 8192
NSTEPS = -(-V // CHUNK)
SC_CORES = 2


def _sc_gather(pred, tcol):
    mesh = plsc.ScalarSubcoreMesh(axis_name="c", num_cores=SC_CORES)
    half = B // SC_CORES

    @functools.partial(
        pl.kernel,
        mesh=mesh,
        out_type=jax.ShapeDtypeStruct((B, SUBL, LANE), jnp.float32),
        scratch_types=[
            pltpu.SMEM((B,), jnp.int32),
            pltpu.SemaphoreType.DMA,
        ],
    )
    def k(p_hbm, s_hbm, o_hbm, s_smem, sem):
        cid = lax.axis_index("c")
        base = cid * half
        pltpu.async_copy(s_hbm, s_smem, sem).wait()

        @pl.loop(0, half)
        def _(i):
            r = base + i
            r8 = pl.multiple_of(r - r % SUBL, SUBL)
            s = pl.multiple_of(s_smem[r], LANE)
            pltpu.make_async_copy(
                p_hbm.at[pl.ds(r8, SUBL), pl.ds(s, LANE)],
                o_hbm.at[r], sem).start()

        @pl.loop(0, half)
        def _(i):
            r = base + i
            r8 = pl.multiple_of(r - r % SUBL, SUBL)
            s = pl.multiple_of(s_smem[r], LANE)
            pltpu.make_async_copy(
                p_hbm.at[pl.ds(r8, SUBL), pl.ds(s, LANE)],
                o_hbm.at[r], sem).wait()

    return k(pred, tcol)


def _count_body(t_ref, g_ref, c_ref, p_ref, o_ref, v_ref, acc_ref):
    step = pl.program_id(0)

    @pl.when(step == 0)
    def _():
        rowid = lax.broadcasted_iota(jnp.int32, (B, SUBL, LANE), 0)
        subl = lax.broadcasted_iota(jnp.int32, (B, SUBL, LANE), 1)
        sel_sub = subl == rowid % SUBL
        picked = jnp.sum(jnp.where(sel_sub, g_ref[...], 0.0), axis=1)
        lane = lax.broadcasted_iota(jnp.int32, (B, LANE), 1)
        v_ref[...] = jnp.sum(jnp.where(lane == c_ref[...], picked, 0.0),
                             axis=1, keepdims=True)
        acc_ref[...] = jnp.zeros_like(acc_ref)

    p = p_ref[...]
    v = v_ref[...]
    lane = lax.broadcasted_iota(jnp.int32, (B, CHUNK), 1)
    tb = t_ref[...] - step * CHUNK
    outranks = (p > v) | ((p == v) & (lane < tb))

    @pl.when(step < NSTEPS - 1)
    def _():
        acc_ref[...] += outranks.astype(jnp.float32)

    @pl.when(step == NSTEPS - 1)
    def _():
        valid = lane < (V - (NSTEPS - 1) * CHUNK)
        acc = acc_ref[...] + (outranks & valid).astype(jnp.float32)
        rank = jnp.sum(acc, axis=1, keepdims=True)
        hits = jnp.sum((rank < float(TOPK_K)).astype(jnp.float32),
                       axis=0, keepdims=True)
        o_ref[...] = hits * (100.0 / B)


def _count(prediction, t, g, c):
    return pl.pallas_call(
        _count_body,
        grid=(NSTEPS,),
        in_specs=[
            pl.BlockSpec((B, 1), lambda i: (0, 0)),
            pl.BlockSpec((B, SUBL, LANE), lambda i: (0, 0, 0)),
            pl.BlockSpec((B, 1), lambda i: (0, 0)),
            pl.BlockSpec((B, CHUNK), lambda i: (0, i)),
        ],
        out_specs=pl.BlockSpec((1, 1), lambda i: (0, 0)),
        out_shape=jax.ShapeDtypeStruct((1, 1), jnp.float32),
        scratch_shapes=[
            pltpu.VMEM((B, 1), jnp.float32),
            pltpu.VMEM((B, CHUNK), jnp.float32),
        ],
    )(t, g, c, prediction)


def kernel(prediction, target):
    target = target.astype(jnp.int32)
    t = target.reshape(B, 1)
    tcol = (target // LANE * LANE).astype(jnp.int32)
    lanes = (target % LANE).reshape(B, 1)
    g = _sc_gather(prediction, tcol)
    res = _count(prediction, t, g, lanes)
    return res[0, 0]


# speedup vs baseline: 2.1310x; 1.3635x over previous
import jax
import jax.numpy as jnp
from jax import lax
from jax.experimental import pallas as pl
from jax.experimental.pallas import tpu as pltpu

B = 128
V = 100000
TOPK_K = 5
LANE = 128
CHUNK = 8192
NSTEPS = -(-V // CHUNK)
LAST_BASE = (NSTEPS - 1) * CHUNK
TAIL = V - LAST_BASE


def _count_body(ts_ref, t_ref, ph_ref, p_ref, o_ref, v_ref,
                acc_ref, g_ref, sem):
    step = pl.program_id(0)
    lane = lax.broadcasted_iota(jnp.int32, (B, CHUNK), 1)
    p = p_ref[...]

    @pl.when(step == 0)
    def _():
        @pl.loop(0, B)
        def _(i):
            t = ts_ref[i]
            tc = pl.multiple_of(
                jnp.minimum(t - t % LANE, LAST_BASE - LANE), LANE)
            pltpu.make_async_copy(ph_ref.at[i].at[pl.ds(tc, LANE)],
                                  g_ref.at[i], sem).start()
        tv = t_ref[...]
        selb = lane == tv - LAST_BASE
        vblk = jnp.sum(jnp.where(selb, p, 0.0), axis=1, keepdims=True)
        pltpu.make_async_copy(ph_ref.at[slice(0, B), pl.ds(0, LANE)],
                              g_ref, sem).wait()
        lane128 = lax.broadcasted_iota(jnp.int32, (B, LANE), 1)
        tcv = jnp.minimum(tv - tv % LANE, LAST_BASE - LANE)
        vg = jnp.sum(jnp.where(lane128 == tv - tcv, g_ref[...], 0.0),
                     axis=1, keepdims=True)
        v_ref[...] = vblk + vg

    bstart = jnp.where(step == 0, LAST_BASE, (step - 1) * CHUNK)
    tb = t_ref[...] - bstart
    v = v_ref[...]
    m = (p > v) | ((p == v) & (lane < tb))

    @pl.when(step == 0)
    def _():
        acc_ref[...] = jnp.sum((m & (lane < TAIL)).astype(jnp.float32),
                               axis=1, keepdims=True)

    @pl.when(jnp.logical_and(step > 0, step < NSTEPS - 1))
    def _():
        acc_ref[...] += jnp.sum(m.astype(jnp.float32), axis=1,
                                keepdims=True)

    @pl.when(step == NSTEPS - 1)
    def _():
        rank = acc_ref[...] + jnp.sum(m.astype(jnp.float32), axis=1,
                                      keepdims=True)
        hits = jnp.sum((rank < float(TOPK_K)).astype(jnp.float32),
                       axis=0, keepdims=True)
        o_ref[...] = hits * (100.0 / B)


def _count(prediction, target, t):
    return pl.pallas_call(
        _count_body,
        grid=(NSTEPS,),
        in_specs=[
            pl.BlockSpec(memory_space=pltpu.SMEM),
            pl.BlockSpec((B, 1), lambda i: (0, 0)),
            pl.BlockSpec(memory_space=pl.ANY),
            pl.BlockSpec((B, CHUNK),
                         lambda i: (0, (i + NSTEPS - 1) % NSTEPS)),
        ],
        out_specs=pl.BlockSpec((1, 1), lambda i: (0, 0)),
        out_shape=jax.ShapeDtypeStruct((1, 1), jnp.float32),
        scratch_shapes=[
            pltpu.VMEM((B, 1), jnp.float32),
            pltpu.VMEM((B, 1), jnp.float32),
            pltpu.VMEM((B, LANE), jnp.float32),
            pltpu.SemaphoreType.DMA,
        ],
    )(target, t, prediction, prediction)


def kernel(prediction, target):
    target = target.astype(jnp.int32)
    t = target.reshape(B, 1)
    res = _count(prediction, target, t)
    return res[0, 0]
